# baseline (speedup 1.0000x reference)
import functools

import jax
import jax.numpy as jnp
import numpy as np
from jax import lax
from jax.experimental import pallas as pl
from jax.experimental.pallas import tpu as pltpu
from jax.experimental.pallas import tpu_sc as plsc

NM = 512
NC = NM * (NM - 1) // 2
LP = 20
H = 64
HID = 128
NPOS = NM * NM

P = 64
N_CHUNKS = NC // P
N_TILES = 32
CHUNKS_PER_TILE = 64
KH = LP // 2

_row_of_pair = np.repeat(np.arange(1, NM), np.arange(1, NM))
_off = (np.arange(NM) * (np.arange(NM) - 1)) // 2
_SCAT_POS = (NM * _row_of_pair
             + (np.arange(NC) - _off[_row_of_pair])).astype(np.int32) \
    .reshape(N_CHUNKS, 1, P)


def _sc_pool_body(emb_p_hbm, idx_p_hbm, scat_hbm, emb_a_hbm, idx_a_hbm,
                  hp_hbm, ha_hbm, raw_v, idx_v, buf_a, buf_b, acc_v, scat_v,
                  sem_a, sem_b):
    wid = lax.axis_index("s") * 2 + lax.axis_index("c")

    def transpose_idx(slot):
        for j0 in range(P // 16):
            pj = lax.iota(jnp.int32, 16) + (j0 * 16)
            for k in range(LP):
                fk = jnp.full((16,), k, jnp.int32)
                idx_v[slot, k, pl.ds(j0 * 16, 16)] = plsc.load_gather(
                    raw_v.at[slot], [pj, fk])

    def accum(buf, init):
        @pl.loop(0, P)
        def _(j):
            for c4 in range(H // 16):
                sl = pl.ds(c4 * 16, 16)
                v = buf[0, j, sl]
                for k in range(1, KH):
                    v = v + buf[k, j, sl]
                if not init:
                    v = v + acc_v[j, sl]
                acc_v[j, sl] = v

    @pl.when(wid < NM // P)
    def _():
        pltpu.sync_copy(idx_a_hbm.at[wid], raw_v.at[0])
        transpose_idx(0)
        for half, buf in ((0, buf_a), (1, buf_b)):
            for k in range(KH):
                pltpu.async_copy(
                    emb_a_hbm.at[idx_v.at[0, half * KH + k]], buf.at[k],
                    sem_a).wait()
            accum(buf, init=(half == 0))
        pltpu.sync_copy(acc_v, ha_hbm.at[pl.ds(wid * P, P)])

    def fire(buf, idx_slot, half, sem):
        for k in range(KH):
            pltpu.async_copy(
                emb_p_hbm.at[idx_v.at[idx_slot, half * KH + k]], buf.at[k],
                sem)

    def drain(buf, sem):
        for k in range(KH):
            pltpu.make_async_copy(
                emb_p_hbm.at[pl.ds(0, P)], buf.at[k], sem).wait()

    first = wid * CHUNKS_PER_TILE
    cnt = jnp.minimum(N_CHUNKS - first, CHUNKS_PER_TILE)

    @pl.when(cnt > 0)
    def _():
        pltpu.sync_copy(idx_p_hbm.at[first], raw_v.at[0])
        transpose_idx(0)
        fire(buf_a, 0, 0, sem_a)

    @pl.loop(0, CHUNKS_PER_TILE)
    def _(g):
        @pl.when(g < cnt)
        def _():
            chunk = first + g
            s = lax.rem(g, 2)
            drain(buf_a, sem_a)
            fire(buf_b, s, 1, sem_b)
            accum(buf_a, init=True)
            drain(buf_b, sem_b)

            @pl.when(g < cnt - 1)
            def _():
                pltpu.sync_copy(idx_p_hbm.at[chunk + 1], raw_v.at[1 - s])
                transpose_idx(1 - s)
                fire(buf_a, 1 - s, 0, sem_a)

            accum(buf_b, init=False)
            pltpu.sync_copy(scat_hbm.at[chunk], scat_v)
            pltpu.sync_copy(acc_v, hp_hbm.at[scat_v.at[0]])


def _sc_pool(emb_p, idx_p, scat_pos, emb_a, idx_a):
    mesh = plsc.VectorSubcoreMesh(core_axis_name="c", subcore_axis_name="s")
    kern = pl.kernel(
        _sc_pool_body,
        out_type=[
            jax.ShapeDtypeStruct((NPOS, H), jnp.float32),
            jax.ShapeDtypeStruct((NM, H), jnp.float32),
        ],
        mesh=mesh,
        scratch_types=[
            pltpu.VMEM((2, P, LP), jnp.int32),
            pltpu.VMEM((2, LP, P), jnp.int32),
            pltpu.VMEM((KH, P, H), jnp.float32),
            pltpu.VMEM((KH, P, H), jnp.float32),
            pltpu.VMEM((P, H), jnp.float32),
            pltpu.VMEM((1, P), jnp.int32),
            pltpu.SemaphoreType.DMA,
            pltpu.SemaphoreType.DMA,
        ],
        compiler_params=pltpu.CompilerParams(use_tc_tiling_on_sc=False,
                                             needs_layout_passes=False),
    )
    return kern(emb_p, idx_p, scat_pos, emb_a, idx_a)


def _tc_score_body(hp_ref, ha_ref, bias_a_ref, bias_p_ref, w1at_ref, w1pt_ref,
                   b1_ref, we_ref, be_ref, w2_ref, b2_ref, out_ref,
                   a_scr, eps_scr):
    b = pl.program_id(0)

    @pl.when(b == 0)
    def _():
        h_a = jnp.tanh(ha_ref[...] + bias_a_ref[...])
        a_scr[...] = jnp.dot(h_a, w1at_ref[...],
                             preferred_element_type=jnp.float32) + b1_ref[...]
        eps_scr[...] = jnp.dot(we_ref[...], h_a.T,
                               preferred_element_type=jnp.float32) + be_ref[...]

    hp = jnp.tanh(hp_ref[...] + bias_p_ref[...])
    hp2 = jnp.dot(hp, w1pt_ref[...], preferred_element_type=jnp.float32)

    for t in range(8):
        r = b * 8 + t
        a_row = a_scr[pl.ds(r, 1), :]
        hid = jnp.tanh(hp2[t * NM:(t + 1) * NM, :] + a_row)
        ana = jnp.dot(w2_ref[...], hid.T,
                      preferred_element_type=jnp.float32) + b2_ref[...]
        jl = lax.broadcasted_iota(jnp.int32, (1, NM), 1)
        row = (jnp.where(jl < r, ana, 0.0)
               + jnp.where(jl == r, eps_scr[...], 0.0))
        out_ref[pl.ds(t, 1), :] = row


def _tc_score(hp_pad, ha_pre, bias_a, bias_p, w1at, w1pt, b1, we, be, w2, b2):
    rep = lambda shape: pl.BlockSpec(shape, lambda b: tuple(0 for _ in shape))
    return pl.pallas_call(
        _tc_score_body,
        grid=(NM // 8,),
        in_specs=[
            pl.BlockSpec((8 * NM, H), lambda b: (b, 0)),
            rep((NM, H)),
            rep((1, H)),
            rep((1, H)),
            rep((H, HID)),
            rep((H, HID)),
            rep((1, HID)),
            rep((1, H)),
            rep((1, 1)),
            rep((1, HID)),
            rep((1, 1)),
        ],
        out_specs=pl.BlockSpec((8, NM), lambda b: (b, 0)),
        out_shape=jax.ShapeDtypeStruct((NM, NM), jnp.float32),
        scratch_shapes=[
            pltpu.VMEM((NM, HID), jnp.float32),
            pltpu.VMEM((1, NM), jnp.float32),
        ],
    )(hp_pad, ha_pre, bias_a, bias_p, w1at, w1pt, b1, we, be, w2, b2)


def kernel(phi_a, all_phi_p, emb_a, bias_a, emb_p, bias_p, W1, b1, W2, b2, We, be):
    idx_p = all_phi_p.astype(jnp.int32).reshape(N_CHUNKS, P, LP)
    idx_a = phi_a.astype(jnp.int32).reshape(NM // P, P, LP)
    scat_pos = jnp.asarray(_SCAT_POS)

    hp_pad, ha_pre = _sc_pool(emb_p, idx_p, scat_pos, emb_a, idx_a)

    scores = _tc_score(
        hp_pad, ha_pre,
        bias_a.reshape(1, H), bias_p.reshape(1, H),
        W1[:, :H].T, W1[:, H:].T,
        b1.reshape(1, HID),
        We.reshape(1, H), be.reshape(1, 1),
        W2.reshape(1, HID), b2.reshape(1, 1),
    )
    return scores


# speedup vs baseline: 1.1559x; 1.1559x over previous
import functools

import jax
import jax.numpy as jnp
import numpy as np
from jax import lax
from jax.experimental import pallas as pl
from jax.experimental.pallas import tpu as pltpu
from jax.experimental.pallas import tpu_sc as plsc

NM = 512
NC = NM * (NM - 1) // 2
LP = 20
H = 64
HID = 128
NPOS = NM * NM

P = 64
N_CHUNKS = NC // P
N_TILES = 32
CHUNKS_PER_TILE = 64
KH = LP // 2

_row_of_pair = np.repeat(np.arange(1, NM), np.arange(1, NM))
_off = (np.arange(NM) * (np.arange(NM) - 1)) // 2
_col = np.arange(NC) - _off[_row_of_pair]
_SCAT_POS = (NM * _row_of_pair
             + np.where(_col < NM // 2, 2 * _col, 2 * _col - (NM - 1))) \
    .astype(np.int32).reshape(N_CHUNKS, 1, P)


def _sc_pool_body(emb_p_hbm, idx_p_hbm, scat_hbm, emb_a_hbm, idx_a_hbm,
                  hp_hbm, ha_hbm, raw_v, idx_v, buf_a, buf_b, acc_v, scat_v,
                  sem_a, sem_b):
    wid = lax.axis_index("s") * 2 + lax.axis_index("c")

    def transpose_idx(slot):
        for j0 in range(P // 16):
            pj = lax.iota(jnp.int32, 16) + (j0 * 16)
            for k in range(LP):
                fk = jnp.full((16,), k, jnp.int32)
                idx_v[slot, k, pl.ds(j0 * 16, 16)] = plsc.load_gather(
                    raw_v.at[slot], [pj, fk])

    def accum(buf, init):
        @pl.loop(0, P)
        def _(j):
            for c4 in range(H // 16):
                sl = pl.ds(c4 * 16, 16)
                v = buf[0, j, sl]
                for k in range(1, KH):
                    v = v + buf[k, j, sl]
                if not init:
                    v = v + acc_v[j, sl]
                acc_v[j, sl] = v

    @pl.when(wid < NM // P)
    def _():
        pltpu.sync_copy(idx_a_hbm.at[pl.ds(wid * P, P)], raw_v.at[0])
        transpose_idx(0)
        for half, buf in ((0, buf_a), (1, buf_b)):
            for k in range(KH):
                pltpu.async_copy(
                    emb_a_hbm.at[idx_v.at[0, half * KH + k]], buf.at[k],
                    sem_a).wait()
            accum(buf, init=(half == 0))
        pltpu.sync_copy(acc_v, ha_hbm.at[pl.ds(wid * P, P)])

    def fire(buf, idx_slot, half, sem):
        for k in range(KH):
            pltpu.async_copy(
                emb_p_hbm.at[idx_v.at[idx_slot, half * KH + k]], buf.at[k],
                sem)

    def drain(buf, sem):
        for k in range(KH):
            pltpu.make_async_copy(
                emb_p_hbm.at[pl.ds(0, P)], buf.at[k], sem).wait()

    first = wid * CHUNKS_PER_TILE
    cnt = jnp.minimum(N_CHUNKS - first, CHUNKS_PER_TILE)

    @pl.when(cnt > 0)
    def _():
        pltpu.sync_copy(idx_p_hbm.at[pl.ds(first * P, P)], raw_v.at[0])
        transpose_idx(0)
        fire(buf_a, 0, 0, sem_a)

    @pl.loop(0, CHUNKS_PER_TILE)
    def _(g):
        @pl.when(g < cnt)
        def _():
            chunk = first + g
            s = lax.rem(g, 2)
            drain(buf_a, sem_a)
            fire(buf_b, s, 1, sem_b)
            accum(buf_a, init=True)
            drain(buf_b, sem_b)

            @pl.when(g < cnt - 1)
            def _():
                pltpu.sync_copy(idx_p_hbm.at[pl.ds((chunk + 1) * P, P)],
                                raw_v.at[1 - s])
                transpose_idx(1 - s)
                fire(buf_a, 1 - s, 0, sem_a)

            accum(buf_b, init=False)
            pltpu.sync_copy(scat_hbm.at[chunk], scat_v)
            pltpu.sync_copy(acc_v, hp_hbm.at[scat_v.at[0]])


def _sc_pool(emb_p, idx_p, scat_pos, emb_a, idx_a):
    mesh = plsc.VectorSubcoreMesh(core_axis_name="c", subcore_axis_name="s")
    kern = pl.kernel(
        _sc_pool_body,
        out_type=[
            jax.ShapeDtypeStruct((NPOS, H), jnp.float32),
            jax.ShapeDtypeStruct((NM, H), jnp.float32),
        ],
        mesh=mesh,
        scratch_types=[
            pltpu.VMEM((2, P, LP), jnp.int32),
            pltpu.VMEM((2, LP, P), jnp.int32),
            pltpu.VMEM((KH, P, H), jnp.float32),
            pltpu.VMEM((KH, P, H), jnp.float32),
            pltpu.VMEM((P, H), jnp.float32),
            pltpu.VMEM((1, P), jnp.int32),
            pltpu.SemaphoreType.DMA,
            pltpu.SemaphoreType.DMA,
        ],
        compiler_params=pltpu.CompilerParams(use_tc_tiling_on_sc=False,
                                             needs_layout_passes=False),
    )
    return kern(emb_p, idx_p, scat_pos, emb_a, idx_a)


def _tc_score_body(hp_ref, ha_ref, bias_a_ref, bias_p2_ref, w1at_ref, w1pt_ref,
                   b1_ref, we_ref, be_ref, w2_ref, b2_ref, out_ref,
                   a_scr, eps_scr):
    b = pl.program_id(0)

    @pl.when(b == 0)
    def _():
        h_a = jnp.tanh(ha_ref[...] + bias_a_ref[...])
        a_scr[...] = jnp.dot(h_a, w1at_ref[...],
                             preferred_element_type=jnp.float32) + b1_ref[...]
        eps_scr[...] = jnp.dot(we_ref[...], h_a.T,
                               preferred_element_type=jnp.float32) + be_ref[...]

    hpt = jnp.tanh(hp_ref[...] + bias_p2_ref[...])
    HM = NM // 2
    hp2e = jnp.dot(hpt[:, :H], w1pt_ref[...], preferred_element_type=jnp.float32)
    hp2o = jnp.dot(hpt[:, H:], w1pt_ref[...], preferred_element_type=jnp.float32)

    for t in range(8):
        r = b * 8 + t
        a_row = a_scr[pl.ds(r, 1), :]
        halves = []
        for half, hp2 in ((0, hp2e), (1, hp2o)):
            hid = jnp.tanh(hp2[t * HM:(t + 1) * HM, :] + a_row)
            ana = jnp.dot(w2_ref[...], hid.T,
                          preferred_element_type=jnp.float32) + b2_ref[...]
            jl = (lax.broadcasted_iota(jnp.int32, (1, HM), 1) + half * HM)
            eps_h = eps_scr[:, half * HM:(half + 1) * HM]
            halves.append(jnp.where(jl < r, ana, 0.0)
                          + jnp.where(jl == r, eps_h, 0.0))
        out_ref[pl.ds(t, 1), :] = jnp.concatenate(halves, axis=1)


def _tc_score(hp_pad, ha_pre, bias_a, bias_p, w1at, w1pt, b1, we, be, w2, b2):
    rep = lambda shape: pl.BlockSpec(shape, lambda b: tuple(0 for _ in shape))
    return pl.pallas_call(
        _tc_score_body,
        grid=(NM // 8,),
        in_specs=[
            pl.BlockSpec((4 * NM, 2 * H), lambda b: (b, 0)),
            rep((NM, H)),
            rep((1, H)),
            rep((1, 2 * H)),
            rep((H, HID)),
            rep((H, HID)),
            rep((1, HID)),
            rep((1, H)),
            rep((1, 1)),
            rep((1, HID)),
            rep((1, 1)),
        ],
        out_specs=pl.BlockSpec((8, NM), lambda b: (b, 0)),
        out_shape=jax.ShapeDtypeStruct((NM, NM), jnp.float32),
        scratch_shapes=[
            pltpu.VMEM((NM, HID), jnp.float32),
            pltpu.VMEM((1, NM), jnp.float32),
        ],
    )(hp_pad, ha_pre, bias_a, bias_p, w1at, w1pt, b1, we, be, w2, b2)


def kernel(phi_a, all_phi_p, emb_a, bias_a, emb_p, bias_p, W1, b1, W2, b2, We, be):
    scat_pos = jnp.asarray(_SCAT_POS)

    hp_pad, ha_pre = _sc_pool(emb_p, all_phi_p, scat_pos, emb_a, phi_a)

    scores = _tc_score(
        hp_pad.reshape(NPOS // 2, 2 * H), ha_pre,
        bias_a.reshape(1, H),
        jnp.concatenate([bias_p, bias_p]).reshape(1, 2 * H),
        W1[:, :H].T, W1[:, H:].T,
        b1.reshape(1, HID),
        We.reshape(1, H), be.reshape(1, 1),
        W2.reshape(1, HID), b2.reshape(1, 1),
    )
    return scores


# speedup vs baseline: 1.3554x; 1.1725x over previous
import jax
import jax.numpy as jnp
import numpy as np
from jax import lax
from jax.experimental import pallas as pl
from jax.experimental.pallas import tpu as pltpu
from jax.experimental.pallas import tpu_sc as plsc

NM = 512
NC = NM * (NM - 1) // 2
LP = 20
H = 64
HID = 128
NPOS = NM * NM

P = 64
N_CHUNKS = NC // P
N_TILES = 32
CHUNKS_PER_TILE = 64
KH = LP // 2

_row_of_pair = np.repeat(np.arange(1, NM), np.arange(1, NM))
_off = (np.arange(NM) * (np.arange(NM) - 1)) // 2
_col = np.arange(NC) - _off[_row_of_pair]
_SCAT_POS = (NM * _row_of_pair
             + np.where(_col < NM // 2, 2 * _col, 2 * _col - (NM - 1))) \
    .astype(np.int32).reshape(N_CHUNKS, 1, P)


def _sc_pool_body(emb_p_hbm, idx_p_hbm, scat_hbm, emb_a_hbm, idx_a_hbm,
                  hp_hbm, ha_hbm, idx_v, buf_a, buf_b, acc_v, scat_v,
                  sem_a, sem_b):
    wid = lax.axis_index("s") * 2 + lax.axis_index("c")

    def accum(buf, init):
        @pl.loop(0, P)
        def _(j):
            for c4 in range(H // 16):
                sl = pl.ds(c4 * 16, 16)
                v = buf[0, j, sl]
                for k in range(1, KH):
                    v = v + buf[k, j, sl]
                if not init:
                    v = v + acc_v[j, sl]
                acc_v[j, sl] = v

    @pl.when(wid < NM // P)
    def _():
        pltpu.sync_copy(idx_a_hbm.at[wid], idx_v.at[0])
        for half, buf in ((0, buf_a), (1, buf_b)):
            for k in range(KH):
                pltpu.async_copy(
                    emb_a_hbm.at[idx_v.at[0, half * KH + k]], buf.at[k],
                    sem_a).wait()
            accum(buf, init=(half == 0))
        pltpu.sync_copy(acc_v, ha_hbm.at[pl.ds(wid * P, P)])

    def fire(buf, idx_slot, half, sem):
        for k in range(KH):
            pltpu.async_copy(
                emb_p_hbm.at[idx_v.at[idx_slot, half * KH + k]], buf.at[k],
                sem)

    def drain(buf, sem):
        for k in range(KH):
            pltpu.make_async_copy(
                emb_p_hbm.at[pl.ds(0, P)], buf.at[k], sem).wait()

    first = wid * CHUNKS_PER_TILE
    cnt = jnp.minimum(N_CHUNKS - first, CHUNKS_PER_TILE)

    @pl.when(cnt > 0)
    def _():
        pltpu.sync_copy(idx_p_hbm.at[first], idx_v.at[0])
        fire(buf_a, 0, 0, sem_a)

    @pl.loop(0, CHUNKS_PER_TILE)
    def _(g):
        @pl.when(g < cnt)
        def _():
            chunk = first + g
            s = lax.rem(g, 2)
            drain(buf_a, sem_a)
            fire(buf_b, s, 1, sem_b)
            accum(buf_a, init=True)
            drain(buf_b, sem_b)

            @pl.when(g < cnt - 1)
            def _():
                pltpu.sync_copy(idx_p_hbm.at[chunk + 1], idx_v.at[1 - s])
                fire(buf_a, 1 - s, 0, sem_a)

            accum(buf_b, init=False)
            pltpu.sync_copy(scat_hbm.at[chunk], scat_v)
            pltpu.sync_copy(acc_v, hp_hbm.at[scat_v.at[0]])


def _sc_pool(emb_p, idx_p, scat_pos, emb_a, idx_a):
    mesh = plsc.VectorSubcoreMesh(core_axis_name="c", subcore_axis_name="s")
    kern = pl.kernel(
        _sc_pool_body,
        out_type=[
            jax.ShapeDtypeStruct((NPOS, H), jnp.float32),
            jax.ShapeDtypeStruct((NM, H), jnp.float32),
        ],
        mesh=mesh,
        scratch_types=[
            pltpu.VMEM((2, LP, P), jnp.int32),
            pltpu.VMEM((KH, P, H), jnp.float32),
            pltpu.VMEM((KH, P, H), jnp.float32),
            pltpu.VMEM((P, H), jnp.float32),
            pltpu.VMEM((1, P), jnp.int32),
            pltpu.SemaphoreType.DMA,
            pltpu.SemaphoreType.DMA,
        ],
        compiler_params=pltpu.CompilerParams(use_tc_tiling_on_sc=False),
    )
    return kern(emb_p, idx_p, scat_pos, emb_a, idx_a)


def _tc_score_body(hp_ref, ha_ref, bias_a_ref, bias_p2_ref, w1at_ref, w1pt_ref,
                   b1_ref, we_ref, be_ref, w2_ref, b2_ref, out_ref,
                   a_scr, eps_scr):
    b = pl.program_id(0)

    @pl.when(b == 0)
    def _():
        h_a = jnp.tanh(ha_ref[...] + bias_a_ref[...])
        a_scr[...] = jnp.dot(h_a, w1at_ref[...],
                             preferred_element_type=jnp.float32) + b1_ref[...]
        eps_scr[...] = jnp.dot(we_ref[...], h_a.T,
                               preferred_element_type=jnp.float32) + be_ref[...]

    hpt = jnp.tanh(hp_ref[...] + bias_p2_ref[...])
    HM = NM // 2
    hp2e = jnp.dot(hpt[:, :H], w1pt_ref[...], preferred_element_type=jnp.float32)
    hp2o = jnp.dot(hpt[:, H:], w1pt_ref[...], preferred_element_type=jnp.float32)

    for t in range(8):
        r = b * 8 + t
        a_row = a_scr[pl.ds(r, 1), :]
        halves = []
        for half, hp2 in ((0, hp2e), (1, hp2o)):
            hid = jnp.tanh(hp2[t * HM:(t + 1) * HM, :] + a_row)
            ana = jnp.dot(w2_ref[...], hid.T,
                          preferred_element_type=jnp.float32) + b2_ref[...]
            jl = (lax.broadcasted_iota(jnp.int32, (1, HM), 1) + half * HM)
            eps_h = eps_scr[:, half * HM:(half + 1) * HM]
            halves.append(jnp.where(jl < r, ana, 0.0)
                          + jnp.where(jl == r, eps_h, 0.0))
        out_ref[pl.ds(t, 1), :] = jnp.concatenate(halves, axis=1)


def _tc_score(hp_pad, ha_pre, bias_a, bias_p, w1at, w1pt, b1, we, be, w2, b2):
    rep = lambda shape: pl.BlockSpec(shape, lambda b: tuple(0 for _ in shape))
    return pl.pallas_call(
        _tc_score_body,
        grid=(NM // 8,),
        in_specs=[
            pl.BlockSpec((4 * NM, 2 * H), lambda b: (b, 0)),
            rep((NM, H)),
            rep((1, H)),
            rep((1, 2 * H)),
            rep((H, HID)),
            rep((H, HID)),
            rep((1, HID)),
            rep((1, H)),
            rep((1, 1)),
            rep((1, HID)),
            rep((1, 1)),
        ],
        out_specs=pl.BlockSpec((8, NM), lambda b: (b, 0)),
        out_shape=jax.ShapeDtypeStruct((NM, NM), jnp.float32),
        scratch_shapes=[
            pltpu.VMEM((NM, HID), jnp.float32),
            pltpu.VMEM((1, NM), jnp.float32),
        ],
    )(hp_pad, ha_pre, bias_a, bias_p, w1at, w1pt, b1, we, be, w2, b2)


def kernel(phi_a, all_phi_p, emb_a, bias_a, emb_p, bias_p, W1, b1, W2, b2, We, be):
    scat_pos = jnp.asarray(_SCAT_POS)
    idx_p = all_phi_p.astype(jnp.int32) \
        .reshape(N_CHUNKS, P, LP).transpose(0, 2, 1)
    idx_a = phi_a.astype(jnp.int32) \
        .reshape(NM // P, P, LP).transpose(0, 2, 1)

    hp_pad, ha_pre = _sc_pool(emb_p, idx_p, scat_pos, emb_a, idx_a)

    scores = _tc_score(
        hp_pad.reshape(NPOS // 2, 2 * H), ha_pre,
        bias_a.reshape(1, H),
        jnp.concatenate([bias_p, bias_p]).reshape(1, 2 * H),
        W1[:, :H].T, W1[:, H:].T,
        b1.reshape(1, HID),
        We.reshape(1, H), be.reshape(1, 1),
        W2.reshape(1, HID), b2.reshape(1, 1),
    )
    return scores
